# baseline (speedup 1.0000x reference)
import functools

import jax
import jax.numpy as jnp
from jax import lax
from jax.experimental import pallas as pl
from jax.experimental.pallas import tpu as pltpu
from jax.experimental.pallas import tpu_sc as plsc

B = 8
N = 2048
BN = B * N
NTASK = 32
RG = 4
L = 16
JV = N // L

SCB = 0
TCB = B - SCB
RT = 128
NT = N // RT


if SCB > 0:
    CH = NTASK // (2 * SCB)
    RW = N // CH
else:
    CH = 1
    RW = N


def _bf16_rne(v):
    c = v * jnp.float32(65537.0)
    return c - (c - v)


def _sc_body(xc, yc, zc, out_hbm,
             qx, qy, qz, qxb, qyb, qzb, rxb, ryb, rzb, rn, outv):
    cid = lax.axis_index("c")
    sid = lax.axis_index("s")
    wid = sid * 2 + cid
    bloc = wid // (2 * CH)
    rem = wid % (2 * CH)
    dirv = rem // CH
    chunk = rem % CH
    b = TCB + bloc

    q_off = (1 - dirv) * BN + b * N + chunk * RW
    r_off = dirv * BN + b * N

    pltpu.sync_copy(xc.at[pl.ds(q_off, RW)], qx.at[pl.ds(0, RW)])
    pltpu.sync_copy(yc.at[pl.ds(q_off, RW)], qy.at[pl.ds(0, RW)])
    pltpu.sync_copy(zc.at[pl.ds(q_off, RW)], qz.at[pl.ds(0, RW)])
    pltpu.sync_copy(xc.at[pl.ds(r_off, N)], rxb)
    pltpu.sync_copy(yc.at[pl.ds(r_off, N)], ryb)
    pltpu.sync_copy(zc.at[pl.ds(r_off, N)], rzb)

    def norm_body(jv, _):
        s = pl.ds(jv * L, L)
        xv = rxb[s]
        yv = ryb[s]
        zv = rzb[s]
        rn[s] = xv * xv + yv * yv + zv * zv
        rxb[s] = _bf16_rne(xv)
        ryb[s] = _bf16_rne(yv)
        rzb[s] = _bf16_rne(zv)
        return 0

    lax.fori_loop(0, JV, norm_body, 0)

    def qround_body(jv, _):
        s = pl.ds(jv * L, L)
        qxb[s] = _bf16_rne(qx[s])
        qyb[s] = _bf16_rne(qy[s])
        qzb[s] = _bf16_rne(qz[s])
        return 0

    lax.fori_loop(0, RW // L, qround_body, 0)

    big = jnp.float32(1e30)
    lanes = lax.iota(jnp.int32, L)

    def lane_min(v):
        for sh in (8, 4, 2, 1):
            v = jnp.minimum(v, v.at[lanes ^ sh].get(
                mode="promise_in_bounds", unique_indices=True))
        return v[0]

    def row_group(g, total):
        i0 = g * RG
        gx = qx[pl.ds(i0, L)]
        gy = qy[pl.ds(i0, L)]
        gz = qz[pl.ds(i0, L)]
        gxb = qxb[pl.ds(i0, L)]
        gyb = qyb[pl.ds(i0, L)]
        gzb = qzb[pl.ds(i0, L)]
        qs = []
        for r in range(RG):
            x = gx[r]
            y = gy[r]
            z = gz[r]
            c = x * x + y * y + z * z
            qs.append((c, -2.0 * gxb[r], -2.0 * gyb[r], -2.0 * gzb[r]))

        def jv_body(jv, mins):
            s = pl.ds(jv * L, L)
            nv = rn[s]
            xv = rxb[s]
            yv = ryb[s]
            zv = rzb[s]
            out = []
            for r in range(RG):
                c, sx, sy, sz = qs[r]
                t = nv + c
                t = t + xv * sx
                t = t + yv * sy
                t = t + zv * sz
                out.append(jnp.minimum(mins[r], t))
            return tuple(out)

        mins = lax.fori_loop(0, JV, jv_body,
                             tuple(jnp.full((L,), big, jnp.float32)
                                   for _ in range(RG)))
        for r in range(RG):
            total = total + jnp.maximum(lane_min(mins[r]), 0.0)
        return total

    total = lax.fori_loop(0, RW // RG, row_group, jnp.float32(0.0))
    outv[...] = jnp.full((L,), total * (1.0 / L), jnp.float32)
    pltpu.sync_copy(outv, out_hbm.at[wid])


def _sc_partials(cat):
    mesh = plsc.VectorSubcoreMesh(core_axis_name="c", subcore_axis_name="s")
    run = functools.partial(
        pl.kernel,
        out_type=jax.ShapeDtypeStruct((NTASK, L), jnp.float32),
        mesh=mesh,
        scratch_types=[
            pltpu.VMEM((RW + L,), jnp.float32),
            pltpu.VMEM((RW + L,), jnp.float32),
            pltpu.VMEM((RW + L,), jnp.float32),
            pltpu.VMEM((RW + L,), jnp.float32),
            pltpu.VMEM((RW + L,), jnp.float32),
            pltpu.VMEM((RW + L,), jnp.float32),
            pltpu.VMEM((N,), jnp.float32),
            pltpu.VMEM((N,), jnp.float32),
            pltpu.VMEM((N,), jnp.float32),
            pltpu.VMEM((N,), jnp.float32),
            pltpu.VMEM((L,), jnp.float32),
        ],
    )(_sc_body)
    return run(cat[0], cat[1], cat[2])


def _tc_body(lhs_ref, rhs_ref, rn_ref, out_ref):
    def batch_body(b, acc):
        rhs = rhs_ref[b]
        rn = rn_ref[b]
        d1sums = jnp.zeros((RT,), jnp.float32)
        d2acc = None
        for i in range(NT):
            lhs = lhs_ref[b, pl.ds(i * RT, RT), :]
            d = lax.dot_general(lhs, rhs, (((1,), (1,)), ((), ())),
                                preferred_element_type=jnp.float32)
            d = d + rn
            macc = d[:, 0:RT]
            for c in range(1, NT):
                macc = jnp.minimum(macc, d[:, c * RT:(c + 1) * RT])
            m1 = jnp.min(macc, axis=1)
            d1sums = d1sums + jnp.maximum(m1, 0.0)
            m2 = jnp.min(d.reshape(NT, 8, N), axis=0)
            d2acc = m2 if i == 0 else jnp.minimum(d2acc, m2)

        d2 = jnp.min(d2acc, axis=0)
        return acc + jnp.sum(d1sums) + jnp.sum(jnp.maximum(d2, 0.0))

    out_ref[0, 0] = lax.fori_loop(0, TCB, batch_body, jnp.float32(0.0))


def _tc_total(lhsb, rhsb, rnq):
    out = pl.pallas_call(
        _tc_body,
        out_specs=pl.BlockSpec(memory_space=pltpu.SMEM),
        out_shape=jax.ShapeDtypeStruct((1, 1), jnp.float32),
    )(lhsb, rhsb, rnq)
    return out[0, 0]


def kernel(array1, array2):
    parts = []

    if TCB > 0:
        a2t = array2[:TCB]
        a1t = array1[:TCB]
        qn = jnp.sum(a2t * a2t, axis=-1)
        rnq = jnp.sum(a1t * a1t, axis=-1)[:, None, :]
        qb = a2t.astype(jnp.bfloat16) * jnp.bfloat16(-2)
        rb = a1t.astype(jnp.bfloat16)
        qn_hi = qn.astype(jnp.bfloat16)
        qn_lo = (qn - qn_hi.astype(jnp.float32)).astype(jnp.bfloat16)
        zero3 = jnp.zeros((TCB, N, 3), jnp.bfloat16)
        one2 = jnp.ones((TCB, N, 2), jnp.bfloat16)
        lhsb = jnp.concatenate(
            [qb, qn_hi[..., None], qn_lo[..., None], zero3], axis=-1)
        rhsb = jnp.concatenate([rb, one2, zero3], axis=-1)
        parts.append(_tc_total(lhsb, rhsb, rnq))

    if SCB > 0:
        c1 = jnp.transpose(array1, (2, 0, 1)).reshape(3, BN)
        c2 = jnp.transpose(array2, (2, 0, 1)).reshape(3, BN)
        cat = jnp.concatenate([c1, c2], axis=1)
        parts.append(jnp.sum(_sc_partials(cat)))

    return sum(parts) / jnp.float32(B * N)


# speedup vs baseline: 2.2384x; 2.2384x over previous
import functools

import jax
import jax.numpy as jnp
from jax import lax
from jax.experimental import pallas as pl
from jax.experimental.pallas import tpu as pltpu
from jax.experimental.pallas import tpu_sc as plsc

B = 8
N = 2048
BN = B * N
NTASK = 32
RG = 4
L = 16
JV = N // L

SCB = 0
TCB = B - SCB
RT = 128
NT = N // RT


if SCB > 0:
    CH = NTASK // (2 * SCB)
    RW = N // CH
else:
    CH = 1
    RW = N


def _bf16_rne(v):
    c = v * jnp.float32(65537.0)
    return c - (c - v)


def _sc_body(xc, yc, zc, out_hbm,
             qx, qy, qz, qxb, qyb, qzb, rxb, ryb, rzb, rn, outv):
    cid = lax.axis_index("c")
    sid = lax.axis_index("s")
    wid = sid * 2 + cid
    bloc = wid // (2 * CH)
    rem = wid % (2 * CH)
    dirv = rem // CH
    chunk = rem % CH
    b = TCB + bloc

    q_off = (1 - dirv) * BN + b * N + chunk * RW
    r_off = dirv * BN + b * N

    pltpu.sync_copy(xc.at[pl.ds(q_off, RW)], qx.at[pl.ds(0, RW)])
    pltpu.sync_copy(yc.at[pl.ds(q_off, RW)], qy.at[pl.ds(0, RW)])
    pltpu.sync_copy(zc.at[pl.ds(q_off, RW)], qz.at[pl.ds(0, RW)])
    pltpu.sync_copy(xc.at[pl.ds(r_off, N)], rxb)
    pltpu.sync_copy(yc.at[pl.ds(r_off, N)], ryb)
    pltpu.sync_copy(zc.at[pl.ds(r_off, N)], rzb)

    def norm_body(jv, _):
        s = pl.ds(jv * L, L)
        xv = rxb[s]
        yv = ryb[s]
        zv = rzb[s]
        rn[s] = xv * xv + yv * yv + zv * zv
        rxb[s] = _bf16_rne(xv)
        ryb[s] = _bf16_rne(yv)
        rzb[s] = _bf16_rne(zv)
        return 0

    lax.fori_loop(0, JV, norm_body, 0)

    def qround_body(jv, _):
        s = pl.ds(jv * L, L)
        qxb[s] = _bf16_rne(qx[s])
        qyb[s] = _bf16_rne(qy[s])
        qzb[s] = _bf16_rne(qz[s])
        return 0

    lax.fori_loop(0, RW // L, qround_body, 0)

    big = jnp.float32(1e30)
    lanes = lax.iota(jnp.int32, L)

    def lane_min(v):
        for sh in (8, 4, 2, 1):
            v = jnp.minimum(v, v.at[lanes ^ sh].get(
                mode="promise_in_bounds", unique_indices=True))
        return v[0]

    def row_group(g, total):
        i0 = g * RG
        gx = qx[pl.ds(i0, L)]
        gy = qy[pl.ds(i0, L)]
        gz = qz[pl.ds(i0, L)]
        gxb = qxb[pl.ds(i0, L)]
        gyb = qyb[pl.ds(i0, L)]
        gzb = qzb[pl.ds(i0, L)]
        qs = []
        for r in range(RG):
            x = gx[r]
            y = gy[r]
            z = gz[r]
            c = x * x + y * y + z * z
            qs.append((c, -2.0 * gxb[r], -2.0 * gyb[r], -2.0 * gzb[r]))

        def jv_body(jv, mins):
            s = pl.ds(jv * L, L)
            nv = rn[s]
            xv = rxb[s]
            yv = ryb[s]
            zv = rzb[s]
            out = []
            for r in range(RG):
                c, sx, sy, sz = qs[r]
                t = nv + c
                t = t + xv * sx
                t = t + yv * sy
                t = t + zv * sz
                out.append(jnp.minimum(mins[r], t))
            return tuple(out)

        mins = lax.fori_loop(0, JV, jv_body,
                             tuple(jnp.full((L,), big, jnp.float32)
                                   for _ in range(RG)))
        for r in range(RG):
            total = total + jnp.maximum(lane_min(mins[r]), 0.0)
        return total

    total = lax.fori_loop(0, RW // RG, row_group, jnp.float32(0.0))
    outv[...] = jnp.full((L,), total * (1.0 / L), jnp.float32)
    pltpu.sync_copy(outv, out_hbm.at[wid])


def _sc_partials(cat):
    mesh = plsc.VectorSubcoreMesh(core_axis_name="c", subcore_axis_name="s")
    run = functools.partial(
        pl.kernel,
        out_type=jax.ShapeDtypeStruct((NTASK, L), jnp.float32),
        mesh=mesh,
        scratch_types=[
            pltpu.VMEM((RW + L,), jnp.float32),
            pltpu.VMEM((RW + L,), jnp.float32),
            pltpu.VMEM((RW + L,), jnp.float32),
            pltpu.VMEM((RW + L,), jnp.float32),
            pltpu.VMEM((RW + L,), jnp.float32),
            pltpu.VMEM((RW + L,), jnp.float32),
            pltpu.VMEM((N,), jnp.float32),
            pltpu.VMEM((N,), jnp.float32),
            pltpu.VMEM((N,), jnp.float32),
            pltpu.VMEM((N,), jnp.float32),
            pltpu.VMEM((L,), jnp.float32),
        ],
    )(_sc_body)
    return run(cat[0], cat[1], cat[2])


def _tc_body(lhs_ref, rhs_ref, rn_ref, out_ref):
    def batch_body(b, acc):
        rhs = rhs_ref[b]
        rn = rn_ref[b]
        d1sums = jnp.zeros((RT,), jnp.float32)
        d2acc = None
        for i in range(NT):
            lhs = lhs_ref[b, :, pl.ds(i * RT, RT)]
            d = lax.dot_general(lhs, rhs, (((0,), (0,)), ((), ())),
                                preferred_element_type=jnp.float32)
            d = d + rn
            macc = d[:, 0:RT]
            for c in range(1, NT):
                macc = jnp.minimum(macc, d[:, c * RT:(c + 1) * RT])
            m1 = jnp.min(macc, axis=1)
            d1sums = d1sums + jnp.maximum(m1, 0.0)
            m2 = jnp.min(d.reshape(NT, 8, N), axis=0)
            d2acc = m2 if i == 0 else jnp.minimum(d2acc, m2)

        d2 = jnp.min(d2acc, axis=0)
        return acc + jnp.sum(d1sums) + jnp.sum(jnp.maximum(d2, 0.0))

    out_ref[0, 0] = lax.fori_loop(0, TCB, batch_body, jnp.float32(0.0))


def _tc_total(lhsb, rhsb, rnq):
    out = pl.pallas_call(
        _tc_body,
        out_specs=pl.BlockSpec(memory_space=pltpu.SMEM),
        out_shape=jax.ShapeDtypeStruct((1, 1), jnp.float32),
    )(lhsb, rhsb, rnq)
    return out[0, 0]


def kernel(array1, array2):
    parts = []

    if TCB > 0:
        a2t = array2[:TCB]
        a1t = array1[:TCB]
        qn = jnp.sum(a2t * a2t, axis=-1)
        rnq = jnp.sum(a1t * a1t, axis=-1)[:, None, :]
        qbT = jnp.transpose(a2t.astype(jnp.bfloat16) * jnp.bfloat16(-2),
                            (0, 2, 1))
        rbT = jnp.transpose(a1t.astype(jnp.bfloat16), (0, 2, 1))
        qn_hi = qn.astype(jnp.bfloat16)
        qn_lo = (qn - qn_hi.astype(jnp.float32)).astype(jnp.bfloat16)
        zero3 = jnp.zeros((TCB, 3, N), jnp.bfloat16)
        one2 = jnp.ones((TCB, 2, N), jnp.bfloat16)
        lhsb = jnp.concatenate(
            [qbT, qn_hi[:, None, :], qn_lo[:, None, :], zero3], axis=1)
        rhsb = jnp.concatenate([rbT, one2, zero3], axis=1)
        parts.append(_tc_total(lhsb, rhsb, rnq))

    if SCB > 0:
        c1 = jnp.transpose(array1, (2, 0, 1)).reshape(3, BN)
        c2 = jnp.transpose(array2, (2, 0, 1)).reshape(3, BN)
        cat = jnp.concatenate([c1, c2], axis=1)
        parts.append(jnp.sum(_sc_partials(cat)))

    return sum(parts) / jnp.float32(B * N)
